# Initial kernel scaffold; baseline (speedup 1.0000x reference)
import jax
import jax.numpy as jnp
from jax.experimental import pallas as pl


def kernel(x, table):
    raise NotImplementedError("write your pallas kernel here")


# speedup vs baseline: 1.6276x; 1.6276x over previous
import functools

import jax
import jax.numpy as jnp
from jax import lax
from jax.experimental import pallas as pl
from jax.experimental.pallas import tpu as pltpu
from jax.experimental.pallas import tpu_sc as plsc


@functools.lru_cache(maxsize=None)
def _make_gather(V, D, B):
  info = plsc.get_sparse_core_info()
  NC, NS = info.num_cores, info.num_subcores
  NW = NC * NS
  assert B % NW == 0
  b_per_w = B // NW
  K = 32
  assert b_per_w % K == 0
  n_chunks = b_per_w // K
  mesh = plsc.VectorSubcoreMesh(core_axis_name="c", subcore_axis_name="s")

  @functools.partial(
      pl.kernel,
      mesh=mesh,
      out_type=jax.ShapeDtypeStruct((B, D), jnp.float32),
      scratch_types=[
          pltpu.VMEM((b_per_w,), jnp.int32),
          pltpu.VMEM((K, D), jnp.float32),
          pltpu.SemaphoreType.DMA,
      ],
  )
  def k(idx_hbm, table_hbm, out_hbm, idx_v, buf, sem):
    wid = lax.axis_index("s") * NC + lax.axis_index("c")
    base = wid * b_per_w
    pltpu.sync_copy(idx_hbm.at[pl.ds(base, b_per_w)], idx_v)

    def body(c, carry):
      off = c * K
      pltpu.async_copy(
          table_hbm.at[idx_v.at[pl.ds(off, K)]], buf, sem
      ).wait()
      pltpu.sync_copy(buf, out_hbm.at[pl.ds(base + off, K)])
      return carry

    lax.fori_loop(0, n_chunks, body, 0)

  return k


def kernel(x, table):
  V, D = table.shape
  B = x.size
  flat_idx = x.reshape((B,)).astype(jnp.int32)
  out = _make_gather(V, D, B)(flat_idx, table)
  return out.reshape(x.shape + (D,))


# speedup vs baseline: 1.7559x; 1.0789x over previous
import functools

import jax
import jax.numpy as jnp
from jax import lax
from jax.experimental import pallas as pl
from jax.experimental.pallas import tpu as pltpu
from jax.experimental.pallas import tpu_sc as plsc


@functools.lru_cache(maxsize=None)
def _make_gather(V, D, B):
  info = plsc.get_sparse_core_info()
  NC, NS = info.num_cores, info.num_subcores
  NW = NC * NS
  assert B % NW == 0
  b_per_w = B // NW
  K = 16
  assert b_per_w % (2 * K) == 0
  n_chunks = b_per_w // K
  mesh = plsc.VectorSubcoreMesh(core_axis_name="c", subcore_axis_name="s")

  @functools.partial(
      pl.kernel,
      mesh=mesh,
      out_type=jax.ShapeDtypeStruct((B, D), jnp.float32),
      scratch_types=[
          pltpu.VMEM((b_per_w,), jnp.int32),
          pltpu.VMEM((K, D), jnp.float32),
          pltpu.VMEM((K, D), jnp.float32),
          pltpu.SemaphoreType.DMA,
          pltpu.SemaphoreType.DMA,
      ],
  )
  def k(idx_hbm, table_hbm, out_hbm, idx_v, buf0, buf1, sem0, sem1):
    wid = lax.axis_index("s") * NC + lax.axis_index("c")
    base = wid * b_per_w
    pltpu.sync_copy(idx_hbm.at[pl.ds(base, b_per_w)], idx_v)

    def gather(off, buf, sem):
      pltpu.async_copy(table_hbm.at[idx_v.at[pl.ds(off, K)]], buf, sem)

    def wait(buf, sem):
      pltpu.make_async_copy(table_hbm.at[pl.ds(0, K)], buf, sem).wait()

    gather(0, buf0, sem0)

    def body(i, carry):
      a = 2 * i
      gather((a + 1) * K, buf1, sem1)
      wait(buf0, sem0)
      pltpu.sync_copy(buf0, out_hbm.at[pl.ds(base + a * K, K)])
      nxt = jnp.minimum((a + 2) * K, (n_chunks - 1) * K)
      gather(nxt, buf0, sem0)
      wait(buf1, sem1)
      pltpu.sync_copy(buf1, out_hbm.at[pl.ds(base + (a + 1) * K, K)])
      return carry

    lax.fori_loop(0, n_chunks // 2, body, 0)
    wait(buf0, sem0)

  return k


def kernel(x, table):
  V, D = table.shape
  B = x.size
  flat_idx = x.reshape((B,)).astype(jnp.int32)
  out = _make_gather(V, D, B)(flat_idx, table)
  return out.reshape(x.shape + (D,))
